# Initial kernel scaffold; baseline (speedup 1.0000x reference)
import jax
import jax.numpy as jnp
from jax.experimental import pallas as pl


def kernel(score, bbox_delta, img_info):
    raise NotImplementedError("write your pallas kernel here")


# speedup vs baseline: 15.1328x; 15.1328x over previous
import functools

import jax
import jax.numpy as jnp
import numpy as np
from jax import lax
from jax.experimental import pallas as pl

_FEAT_STRIDE = 16.0
_NMS_THRESH = 0.7
_PRE = 6000
_PRE_PAD = 6144
_POST = 2000
_POST_PAD = 2048
_A = 9
_H, _W = 38, 50
_NPOS = _H * _W
_NANCH = _NPOS * _A
_NANCH_PAD = 17152
_ROWS, _LANES = 48, 128
_CHUNKS = _NANCH_PAD // _LANES


def _base_anchors():
    base = np.array([0.0, 0.0, 15.0, 15.0])
    w = base[2] - base[0] + 1.0
    h = base[3] - base[1] + 1.0
    x_ctr = base[0] + 0.5 * (w - 1.0)
    y_ctr = base[1] + 0.5 * (h - 1.0)
    size = w * h
    ratios = np.array([0.5, 1.0, 2.0])
    scales = np.array([8.0, 16.0, 32.0])
    ws_r = np.round(np.sqrt(size / ratios))
    hs_r = np.round(ws_r * ratios)
    out = []
    for wr, hr in zip(ws_r, hs_r):
        for s in scales:
            ws = wr * s
            hs = hr * s
            out.append([x_ctr - 0.5 * (ws - 1.0), y_ctr - 0.5 * (hs - 1.0),
                        x_ctr + 0.5 * (ws - 1.0), y_ctr + 0.5 * (hs - 1.0)])
    return np.array(out, dtype=np.float32)


_BASE = _base_anchors()
_BASE_ROWS = np.zeros((4, 1, _LANES), dtype=np.float32)
_BASE_ROWS[:, 0, :_A] = _BASE.T


def _gather128(row, idx):
    return jnp.take_along_axis(jnp.broadcast_to(row, (_ROWS, _LANES)), idx,
                               axis=1)


def _body(idx_ref, sc_ref, dpl_ref, img_ref, base_ref, out_ref):
    idx = idx_ref[0]
    sc = sc_ref[0]

    a = idx % _A
    pos = idx // _A
    gi = (pos // _W).astype(jnp.float32)
    gj = (pos % _W).astype(jnp.float32)
    cx = gj * _FEAT_STRIDE + 0.5 * _FEAT_STRIDE
    cy = gi * _FEAT_STRIDE + 0.5 * _FEAT_STRIDE
    bx1 = _gather128(base_ref[0, :, :], a) + cx
    by1 = _gather128(base_ref[1, :, :], a) + cy
    bx2 = _gather128(base_ref[2, :, :], a) + cx
    by2 = _gather128(base_ref[3, :, :], a) + cy

    cid = idx // _LANES
    lid = idx % _LANES
    zero = jnp.zeros((_ROWS, _LANES), jnp.float32)

    def gbody(c, carry):
        m = cid == c
        got = [_gather128(dpl_ref[0, d, pl.ds(c, 1), :], lid) for d in range(4)]
        return tuple(jnp.where(m, g, old) for g, old in zip(got, carry))

    dx, dy, dw, dh = lax.fori_loop(0, _CHUNKS, gbody, (zero, zero, zero, zero))

    aw = bx2 - bx1 + 1.0
    ah = by2 - by1 + 1.0
    actx = bx1 + 0.5 * aw
    acty = by1 + 0.5 * ah
    pcx = dx * aw + actx
    pcy = dy * ah + acty
    pw = jnp.exp(dw) * aw
    ph = jnp.exp(dh) * ah
    im_h = img_ref[0, 0, 0] - 1.0
    im_w = img_ref[0, 0, 1] - 1.0
    x1 = jnp.clip(pcx - 0.5 * pw, 0.0, im_w)
    y1 = jnp.clip(pcy - 0.5 * ph, 0.0, im_h)
    x2 = jnp.clip(pcx + 0.5 * pw, 0.0, im_w)
    y2 = jnp.clip(pcy + 0.5 * ph, 0.0, im_h)
    area = (x2 - x1 + 1.0) * (y2 - y1 + 1.0)

    flat = (lax.broadcasted_iota(jnp.int32, (_ROWS, _LANES), 0) * _LANES
            + lax.broadcasted_iota(jnp.int32, (_ROWS, _LANES), 1))
    keep0 = jnp.where(flat < _PRE, 1.0, 0.0)

    def nms_body(i, kp):
        oh = flat == i
        ohf = jnp.where(oh, 1.0, 0.0)
        xi1 = jnp.sum(ohf * x1)
        yi1 = jnp.sum(ohf * y1)
        xi2 = jnp.sum(ohf * x2)
        yi2 = jnp.sum(ohf * y2)
        ai = jnp.sum(ohf * area)
        iw = jnp.maximum(jnp.minimum(x2, xi2) - jnp.maximum(x1, xi1) + 1.0, 0.0)
        ih = jnp.maximum(jnp.minimum(y2, yi2) - jnp.maximum(y1, yi1) + 1.0, 0.0)
        inter = iw * ih
        ov = inter > _NMS_THRESH * (area + ai - inter)
        sup = jnp.max(jnp.where((flat < i) & ov, kp, 0.0)) > 0.0
        return jnp.where(oh & sup, 0.0, kp)

    kf = lax.fori_loop(0, _PRE, nms_body, keep0)
    ut = (lax.broadcasted_iota(jnp.int32, (_LANES, _LANES), 0)
          <= lax.broadcasted_iota(jnp.int32, (_LANES, _LANES), 1))
    csum = lax.dot_general(kf, ut.astype(jnp.float32), (((1,), (0,)), ((), ())),
                           preferred_element_type=jnp.float32)
    rows_sum = csum[:, _LANES - 1:_LANES]
    lt = (lax.broadcasted_iota(jnp.int32, (_ROWS, _ROWS), 0)
          > lax.broadcasted_iota(jnp.int32, (_ROWS, _ROWS), 1))
    off = lax.dot_general(lt.astype(jnp.float32), rows_sum,
                          (((1,), (0,)), ((), ())),
                          preferred_element_type=jnp.float32)
    rank = csum + off - kf

    rank_t = jnp.transpose(rank)
    kt = jnp.transpose(kf)
    sct = jnp.transpose(sc)
    x1t = jnp.transpose(x1)
    y1t = jnp.transpose(y1)
    x2t = jnp.transpose(x2)
    y2t = jnp.transpose(y2)
    oio = lax.broadcasted_iota(jnp.int32, (1, _POST_PAD), 1).astype(jnp.float32)
    acc = jnp.zeros((5, _POST_PAD), jnp.float32)
    for r in range(_ROWS):
        sel = ((rank_t[:, r:r + 1] == oio) & (kt[:, r:r + 1] > 0.0))
        v = jnp.concatenate([sct[:, r:r + 1], x1t[:, r:r + 1], y1t[:, r:r + 1],
                             x2t[:, r:r + 1], y2t[:, r:r + 1]], axis=1)
        acc = acc + lax.dot_general(
            v, sel.astype(jnp.float32), (((0,), (0,)), ((), ())),
            preferred_element_type=jnp.float32,
            precision=lax.Precision.HIGHEST)
    out_ref[0, 0:5, :] = acc


def kernel(score, bbox_delta, img_info):
    B = score.shape[0]
    fg = jnp.transpose(score[:, _A:, :, :], (0, 2, 3, 1)).reshape(B, _NANCH)
    sc, idx = lax.top_k(fg, _PRE)
    idx2d = jnp.pad(idx, ((0, 0), (0, _PRE_PAD - _PRE))).reshape(
        B, _ROWS, _LANES)
    sc2d = jnp.pad(sc, ((0, 0), (0, _PRE_PAD - _PRE))).reshape(
        B, _ROWS, _LANES)
    dpl = jnp.transpose(
        jnp.transpose(bbox_delta, (0, 2, 3, 1)).reshape(B, _NPOS, _A, 4),
        (0, 3, 1, 2)).reshape(B, 4, _NANCH)
    dpl = jnp.pad(dpl, ((0, 0), (0, 0), (0, _NANCH_PAD - _NANCH))).reshape(
        B, 4, _CHUNKS, _LANES)
    img = jnp.pad(img_info.reshape(B, 1, 3), ((0, 0), (0, 0), (0, 125)))

    out = pl.pallas_call(
        _body,
        grid=(B,),
        in_specs=[
            pl.BlockSpec((1, _ROWS, _LANES), lambda b: (b, 0, 0)),
            pl.BlockSpec((1, _ROWS, _LANES), lambda b: (b, 0, 0)),
            pl.BlockSpec((1, 4, _CHUNKS, _LANES), lambda b: (b, 0, 0, 0)),
            pl.BlockSpec((1, 1, _LANES), lambda b: (b, 0, 0)),
            pl.BlockSpec((4, 1, _LANES), lambda b: (0, 0, 0)),
        ],
        out_specs=pl.BlockSpec((1, 8, _POST_PAD), lambda b: (b, 0, 0)),
        out_shape=jax.ShapeDtypeStruct((B, 8, _POST_PAD), jnp.float32),
    )(idx2d, sc2d, dpl, img, jnp.asarray(_BASE_ROWS))

    return jnp.transpose(out[:, 0:5, :_POST], (0, 2, 1))


# speedup vs baseline: 26.5915x; 1.7572x over previous
import functools

import jax
import jax.numpy as jnp
import numpy as np
from jax import lax
from jax.experimental import pallas as pl

_FEAT_STRIDE = 16.0
_NMS_THRESH = 0.7
_PRE = 6000
_PRE_PAD = 6144
_POST = 2000
_POST_PAD = 2048
_A = 9
_H, _W = 38, 50
_NPOS = _H * _W
_NANCH = _NPOS * _A
_NANCH_PAD = 17152
_ROWS, _LANES = 48, 128
_CHUNKS = _NANCH_PAD // _LANES


def _base_anchors():
    base = np.array([0.0, 0.0, 15.0, 15.0])
    w = base[2] - base[0] + 1.0
    h = base[3] - base[1] + 1.0
    x_ctr = base[0] + 0.5 * (w - 1.0)
    y_ctr = base[1] + 0.5 * (h - 1.0)
    size = w * h
    ratios = np.array([0.5, 1.0, 2.0])
    scales = np.array([8.0, 16.0, 32.0])
    ws_r = np.round(np.sqrt(size / ratios))
    hs_r = np.round(ws_r * ratios)
    out = []
    for wr, hr in zip(ws_r, hs_r):
        for s in scales:
            ws = wr * s
            hs = hr * s
            out.append([x_ctr - 0.5 * (ws - 1.0), y_ctr - 0.5 * (hs - 1.0),
                        x_ctr + 0.5 * (ws - 1.0), y_ctr + 0.5 * (hs - 1.0)])
    return np.array(out, dtype=np.float32)


_BASE = _base_anchors()
_BASE_ROWS = np.zeros((4, 1, _LANES), dtype=np.float32)
_BASE_ROWS[:, 0, :_A] = _BASE.T


def _gather128(row, idx):
    return jnp.take_along_axis(jnp.broadcast_to(row, (_ROWS, _LANES)), idx,
                               axis=1)


def _body(idx_ref, sc_ref, dpl_ref, img_ref, base_ref, out_ref):
    idx = idx_ref[0]
    sc = sc_ref[0]

    a = idx % _A
    pos = idx // _A
    gi = (pos // _W).astype(jnp.float32)
    gj = (pos % _W).astype(jnp.float32)
    cx = gj * _FEAT_STRIDE + 0.5 * _FEAT_STRIDE
    cy = gi * _FEAT_STRIDE + 0.5 * _FEAT_STRIDE
    bx1 = _gather128(base_ref[0, :, :], a) + cx
    by1 = _gather128(base_ref[1, :, :], a) + cy
    bx2 = _gather128(base_ref[2, :, :], a) + cx
    by2 = _gather128(base_ref[3, :, :], a) + cy

    cid = idx // _LANES
    lid = idx % _LANES
    zero = jnp.zeros((_ROWS, _LANES), jnp.float32)

    def gbody(c, carry):
        m = cid == c
        got = [_gather128(dpl_ref[0, d, pl.ds(c, 1), :], lid) for d in range(4)]
        return tuple(jnp.where(m, g, old) for g, old in zip(got, carry))

    dx, dy, dw, dh = lax.fori_loop(0, _CHUNKS, gbody, (zero, zero, zero, zero))

    aw = bx2 - bx1 + 1.0
    ah = by2 - by1 + 1.0
    actx = bx1 + 0.5 * aw
    acty = by1 + 0.5 * ah
    pcx = dx * aw + actx
    pcy = dy * ah + acty
    pw = jnp.exp(dw) * aw
    ph = jnp.exp(dh) * ah
    im_h = img_ref[0, 0, 0] - 1.0
    im_w = img_ref[0, 0, 1] - 1.0
    x1 = jnp.clip(pcx - 0.5 * pw, 0.0, im_w)
    y1 = jnp.clip(pcy - 0.5 * ph, 0.0, im_h)
    x2 = jnp.clip(pcx + 0.5 * pw, 0.0, im_w)
    y2 = jnp.clip(pcy + 0.5 * ph, 0.0, im_h)
    area = (x2 - x1 + 1.0) * (y2 - y1 + 1.0)

    flat = (lax.broadcasted_iota(jnp.int32, (_ROWS, _LANES), 0) * _LANES
            + lax.broadcasted_iota(jnp.int32, (_ROWS, _LANES), 1))
    lane = lax.broadcasted_iota(jnp.int32, (1, _LANES), 1)
    row_iota = lax.broadcasted_iota(jnp.int32, (_ROWS, _LANES), 0)
    pad_l = jnp.zeros((_LANES, _LANES - _ROWS), jnp.float32)
    x1t = jnp.concatenate([jnp.transpose(x1), pad_l], axis=1)
    y1t = jnp.concatenate([jnp.transpose(y1), pad_l], axis=1)
    x2t = jnp.concatenate([jnp.transpose(x2), pad_l], axis=1)
    y2t = jnp.concatenate([jnp.transpose(y2), pad_l], axis=1)
    art = jnp.concatenate([jnp.transpose(area), pad_l], axis=1)
    keep = jnp.where(flat < _PRE, 1.0, 0.0)

    for b in range(_ROWS):
        rx1 = x1[b:b + 1]
        ry1 = y1[b:b + 1]
        rx2 = x2[b:b + 1]
        ry2 = y2[b:b + 1]
        rar = area[b:b + 1]
        keep_t = jnp.concatenate([jnp.transpose(keep), pad_l],
                                 axis=1)

        def pbody(p, sup, keep_t=keep_t, rx1=rx1, ry1=ry1, rx2=rx2,
                  ry2=ry2, rar=rar):
            pi = jnp.zeros((_LANES, _LANES), jnp.int32) + p
            cx1 = jnp.take_along_axis(x1t, pi, axis=1)[:, 0:1]
            cy1 = jnp.take_along_axis(y1t, pi, axis=1)[:, 0:1]
            cx2 = jnp.take_along_axis(x2t, pi, axis=1)[:, 0:1]
            cy2 = jnp.take_along_axis(y2t, pi, axis=1)[:, 0:1]
            car = jnp.take_along_axis(art, pi, axis=1)[:, 0:1]
            ck = jnp.take_along_axis(keep_t, pi, axis=1)[:, 0:1]
            iw = jnp.maximum(
                jnp.minimum(cx2, rx2) - jnp.maximum(cx1, rx1) + 1.0, 0.0)
            ih = jnp.maximum(
                jnp.minimum(cy2, ry2) - jnp.maximum(cy1, ry1) + 1.0, 0.0)
            inter = iw * ih
            ov = inter > _NMS_THRESH * (car + rar - inter)
            m = jnp.where(ov, ck, 0.0)
            return jnp.maximum(sup, jnp.max(m, axis=0, keepdims=True))

        sup = jnp.zeros((1, _LANES), jnp.float32)
        if b > 0:
            sup = lax.fori_loop(0, b, pbody, sup)
        alive = keep[b:b + 1] * (1.0 - sup)

        rx1b = jnp.broadcast_to(rx1, (8, _LANES))
        ry1b = jnp.broadcast_to(ry1, (8, _LANES))
        rx2b = jnp.broadcast_to(rx2, (8, _LANES))
        ry2b = jnp.broadcast_to(ry2, (8, _LANES))

        def ibody(l, al, rx1=rx1, ry1=ry1, rx2=rx2, ry2=ry2, rar=rar,
                  rx1b=rx1b, ry1b=ry1b, rx2b=rx2b, ry2b=ry2b):
            li = jnp.zeros((8, _LANES), jnp.int32) + l
            xi1 = jnp.take_along_axis(rx1b, li, axis=1)[0:1]
            yi1 = jnp.take_along_axis(ry1b, li, axis=1)[0:1]
            xi2 = jnp.take_along_axis(rx2b, li, axis=1)[0:1]
            yi2 = jnp.take_along_axis(ry2b, li, axis=1)[0:1]
            alv = jnp.take_along_axis(jnp.broadcast_to(al, (8, _LANES)), li,
                                      axis=1)[0:1]
            ai = (xi2 - xi1 + 1.0) * (yi2 - yi1 + 1.0)
            iw = jnp.maximum(
                jnp.minimum(rx2, xi2) - jnp.maximum(rx1, xi1) + 1.0, 0.0)
            ih = jnp.maximum(
                jnp.minimum(ry2, yi2) - jnp.maximum(ry1, yi1) + 1.0, 0.0)
            inter = iw * ih
            ov = inter > _NMS_THRESH * (rar + ai - inter)
            m = ov & (lane > l) & (alv > 0.0)
            return jnp.where(m, 0.0, al)

        alive = lax.fori_loop(0, _LANES, ibody, alive)
        keep = jnp.where(row_iota == b, jnp.broadcast_to(alive, (_ROWS, _LANES)),
                         keep)
    kf = keep
    ut = (lax.broadcasted_iota(jnp.int32, (_LANES, _LANES), 0)
          <= lax.broadcasted_iota(jnp.int32, (_LANES, _LANES), 1))
    csum = lax.dot_general(kf, ut.astype(jnp.float32), (((1,), (0,)), ((), ())),
                           preferred_element_type=jnp.float32)
    rows_sum = csum[:, _LANES - 1:_LANES]
    lt = (lax.broadcasted_iota(jnp.int32, (_ROWS, _ROWS), 0)
          > lax.broadcasted_iota(jnp.int32, (_ROWS, _ROWS), 1))
    off = lax.dot_general(lt.astype(jnp.float32), rows_sum,
                          (((1,), (0,)), ((), ())),
                          preferred_element_type=jnp.float32)
    rank = csum + off - kf

    rank_t = jnp.transpose(rank)
    kt = jnp.transpose(kf)
    sct = jnp.transpose(sc)
    oio = lax.broadcasted_iota(jnp.int32, (1, _POST_PAD), 1).astype(jnp.float32)
    acc = jnp.zeros((5, _POST_PAD), jnp.float32)
    for r in range(_ROWS):
        sel = ((rank_t[:, r:r + 1] == oio) & (kt[:, r:r + 1] > 0.0))
        v = jnp.concatenate([sct[:, r:r + 1], x1t[:, r:r + 1], y1t[:, r:r + 1],
                             x2t[:, r:r + 1], y2t[:, r:r + 1]], axis=1)
        acc = acc + lax.dot_general(
            v, sel.astype(jnp.float32), (((0,), (0,)), ((), ())),
            preferred_element_type=jnp.float32,
            precision=lax.Precision.HIGHEST)
    out_ref[0, 0:5, :] = acc


def kernel(score, bbox_delta, img_info):
    B = score.shape[0]
    fg = jnp.transpose(score[:, _A:, :, :], (0, 2, 3, 1)).reshape(B, _NANCH)
    sc, idx = lax.top_k(fg, _PRE)
    idx2d = jnp.pad(idx, ((0, 0), (0, _PRE_PAD - _PRE))).reshape(
        B, _ROWS, _LANES)
    sc2d = jnp.pad(sc, ((0, 0), (0, _PRE_PAD - _PRE))).reshape(
        B, _ROWS, _LANES)
    dpl = jnp.transpose(
        jnp.transpose(bbox_delta, (0, 2, 3, 1)).reshape(B, _NPOS, _A, 4),
        (0, 3, 1, 2)).reshape(B, 4, _NANCH)
    dpl = jnp.pad(dpl, ((0, 0), (0, 0), (0, _NANCH_PAD - _NANCH))).reshape(
        B, 4, _CHUNKS, _LANES)
    img = jnp.pad(img_info.reshape(B, 1, 3), ((0, 0), (0, 0), (0, 125)))

    out = pl.pallas_call(
        _body,
        grid=(B,),
        in_specs=[
            pl.BlockSpec((1, _ROWS, _LANES), lambda b: (b, 0, 0)),
            pl.BlockSpec((1, _ROWS, _LANES), lambda b: (b, 0, 0)),
            pl.BlockSpec((1, 4, _CHUNKS, _LANES), lambda b: (b, 0, 0, 0)),
            pl.BlockSpec((1, 1, _LANES), lambda b: (b, 0, 0)),
            pl.BlockSpec((4, 1, _LANES), lambda b: (0, 0, 0)),
        ],
        out_specs=pl.BlockSpec((1, 8, _POST_PAD), lambda b: (b, 0, 0)),
        out_shape=jax.ShapeDtypeStruct((B, 8, _POST_PAD), jnp.float32),
    )(idx2d, sc2d, dpl, img, jnp.asarray(_BASE_ROWS))

    return jnp.transpose(out[:, 0:5, :_POST], (0, 2, 1))
